# Initial kernel scaffold; baseline (speedup 1.0000x reference)
import jax
import jax.numpy as jnp
from jax.experimental import pallas as pl


def kernel(x, batch_index, W_ih, W_hh, b_ih, b_hh):
    raise NotImplementedError("write your pallas kernel here")


# speedup vs baseline: 15.6411x; 15.6411x over previous
import jax
import jax.numpy as jnp
from jax import lax
from jax.experimental import pallas as pl
from jax.experimental.pallas import tpu as pltpu

N = 100000
IN_CH = 128
OUT_CH = 2 * IN_CH
B = 32
STEPS = 3
BLK = 2000
NB = N // BLK

_NEG = -1e30


def _body(x_ref, idx_ref, wi_ref, wh_ref, b_ref, out_ref,
          h_ref, c_ref, qs_ref, m_ref, s_ref, r_ref):
    t = pl.program_id(0)
    b = pl.program_id(1)

    @pl.when(b == 0)
    def _head():
        @pl.when(t == 0)
        def _init():
            h_ref[...] = jnp.zeros((B, IN_CH), jnp.float32)
            c_ref[...] = jnp.zeros((B, IN_CH), jnp.float32)
            qs_ref[...] = jnp.zeros((B, OUT_CH), jnp.float32)

        @pl.when(t > 0)
        def _finalize_prev():
            r = r_ref[...] / (s_ref[...] + 1e-16)
            qs_ref[:, :IN_CH] = h_ref[...]
            qs_ref[:, IN_CH:] = r

        qs = qs_ref[...]
        gates = (jnp.dot(qs, wi_ref[...], preferred_element_type=jnp.float32)
                 + jnp.dot(h_ref[...], wh_ref[...], preferred_element_type=jnp.float32)
                 + b_ref[...])
        ig = jax.nn.sigmoid(gates[:, :IN_CH])
        fg = jax.nn.sigmoid(gates[:, IN_CH:2 * IN_CH])
        gg = jnp.tanh(gates[:, 2 * IN_CH:3 * IN_CH])
        og = jax.nn.sigmoid(gates[:, 3 * IN_CH:])
        c_new = fg * c_ref[...] + ig * gg
        h_new = og * jnp.tanh(c_new)
        c_ref[...] = c_new
        h_ref[...] = h_new
        m_ref[...] = jnp.full((B, 1), _NEG, jnp.float32)
        s_ref[...] = jnp.zeros((B, 1), jnp.float32)
        r_ref[...] = jnp.zeros((B, IN_CH), jnp.float32)

    xb = x_ref[...]
    idx = idx_ref[0]
    q = h_ref[...]

    seg = lax.broadcasted_iota(jnp.int32, (B, BLK), 0)
    mask = idx == seg

    et = lax.dot_general(q, xb, (((1,), (1,)), ((), ())),
                         preferred_element_type=jnp.float32)
    em = jnp.where(mask, et, _NEG)

    m_old = m_ref[...]
    m_blk = jnp.max(em, axis=1, keepdims=True)
    m_new = jnp.maximum(m_old, m_blk)
    scale = jnp.exp(m_old - m_new)

    p = jnp.where(mask, jnp.exp(jnp.minimum(em - m_new, 0.0)), 0.0)

    m_ref[...] = m_new
    s_ref[...] = s_ref[...] * scale + jnp.sum(p, axis=1, keepdims=True)
    r_ref[...] = r_ref[...] * scale + lax.dot_general(
        p, xb, (((1,), (0,)), ((), ())), preferred_element_type=jnp.float32)

    @pl.when((t == STEPS - 1) & (b == NB - 1))
    def _emit():
        out_ref[:, :IN_CH] = h_ref[...]
        out_ref[:, IN_CH:] = r_ref[...] / (s_ref[...] + 1e-16)


def kernel(x, batch_index, W_ih, W_hh, b_ih, b_hh):
    wi = W_ih.T
    wh = W_hh.T
    bias = (b_ih + b_hh).reshape(1, 4 * IN_CH)
    idx3 = batch_index.reshape(NB, 1, BLK)

    return pl.pallas_call(
        _body,
        grid=(STEPS, NB),
        in_specs=[
            pl.BlockSpec((BLK, IN_CH), lambda t, b: (b, 0)),
            pl.BlockSpec((1, 1, BLK), lambda t, b: (b, 0, 0)),
            pl.BlockSpec((OUT_CH, 4 * IN_CH), lambda t, b: (0, 0)),
            pl.BlockSpec((IN_CH, 4 * IN_CH), lambda t, b: (0, 0)),
            pl.BlockSpec((1, 4 * IN_CH), lambda t, b: (0, 0)),
        ],
        out_specs=pl.BlockSpec((B, OUT_CH), lambda t, b: (0, 0)),
        out_shape=jax.ShapeDtypeStruct((B, OUT_CH), jnp.float32),
        scratch_shapes=[
            pltpu.VMEM((B, IN_CH), jnp.float32),
            pltpu.VMEM((B, IN_CH), jnp.float32),
            pltpu.VMEM((B, OUT_CH), jnp.float32),
            pltpu.VMEM((B, 1), jnp.float32),
            pltpu.VMEM((B, 1), jnp.float32),
            pltpu.VMEM((B, IN_CH), jnp.float32),
        ],
        compiler_params=pltpu.CompilerParams(
            dimension_semantics=("arbitrary", "arbitrary"),
        ),
    )(x, idx3, wi, wh, bias)


# speedup vs baseline: 26.0701x; 1.6668x over previous
import jax
import jax.numpy as jnp
from jax import lax
from jax.experimental import pallas as pl
from jax.experimental.pallas import tpu as pltpu

N = 100000
IN_CH = 128
OUT_CH = 2 * IN_CH
B = 32
STEPS = 3
BLK = 5000
NB = N // BLK

_NEG = -1e30


def _body(x_ref, idx_ref, wi_ref, wh_ref, b_ref, out_ref,
          h_ref, c_ref, qs_ref, m_ref, s_ref, r_ref):
    t = pl.program_id(0)
    b = pl.program_id(1)

    @pl.when(b == 0)
    def _head():
        @pl.when(t == 0)
        def _init():
            h_ref[...] = jnp.zeros((B, IN_CH), jnp.float32)
            c_ref[...] = jnp.zeros((B, IN_CH), jnp.float32)
            qs_ref[...] = jnp.zeros((B, OUT_CH), jnp.float32)

        @pl.when(t > 0)
        def _finalize_prev():
            r = jnp.where(m_ref[...] > _NEG,
                          r_ref[...] / (s_ref[...] + 1e-16), 0.0)
            qs_ref[:, :IN_CH] = h_ref[...]
            qs_ref[:, IN_CH:] = r

        qs = qs_ref[...]
        gates = (jnp.dot(qs, wi_ref[...], preferred_element_type=jnp.float32)
                 + jnp.dot(h_ref[...], wh_ref[...], preferred_element_type=jnp.float32)
                 + b_ref[...])
        ig = jax.nn.sigmoid(gates[:, :IN_CH])
        fg = jax.nn.sigmoid(gates[:, IN_CH:2 * IN_CH])
        gg = jnp.tanh(gates[:, 2 * IN_CH:3 * IN_CH])
        og = jax.nn.sigmoid(gates[:, 3 * IN_CH:])
        c_new = fg * c_ref[...] + ig * gg
        h_new = og * jnp.tanh(c_new)
        c_ref[...] = c_new
        h_ref[...] = h_new
        m_ref[...] = jnp.full((B, 1), _NEG, jnp.float32)
        s_ref[...] = jnp.zeros((B, 1), jnp.float32)
        r_ref[...] = jnp.zeros((B, IN_CH), jnp.float32)

    xb = x_ref[...]
    idx = idx_ref[0]
    q = h_ref[...]

    seg = lax.broadcasted_iota(jnp.int32, (B, BLK), 0)
    mask = idx == seg

    et = lax.dot_general(q, xb, (((1,), (1,)), ((), ())),
                         preferred_element_type=jnp.float32)
    em = jnp.where(mask, et, _NEG)

    m_old = m_ref[...]
    m_blk = jnp.max(em, axis=1, keepdims=True)
    m_new = jnp.maximum(m_old, m_blk)
    scale = jnp.exp(m_old - m_new)

    p = jnp.exp(em - m_new)

    m_ref[...] = m_new
    s_ref[...] = s_ref[...] * scale + jnp.sum(p, axis=1, keepdims=True)
    r_ref[...] = r_ref[...] * scale + lax.dot_general(
        p, xb, (((1,), (0,)), ((), ())), preferred_element_type=jnp.float32)

    @pl.when((t == STEPS - 1) & (b == NB - 1))
    def _emit():
        out_ref[:, :IN_CH] = h_ref[...]
        out_ref[:, IN_CH:] = jnp.where(m_ref[...] > _NEG,
                                       r_ref[...] / (s_ref[...] + 1e-16), 0.0)


def kernel(x, batch_index, W_ih, W_hh, b_ih, b_hh):
    wi = W_ih.T
    wh = W_hh.T
    bias = (b_ih + b_hh).reshape(1, 4 * IN_CH)
    idx3 = batch_index.reshape(NB, 1, BLK)

    return pl.pallas_call(
        _body,
        grid=(STEPS, NB),
        in_specs=[
            pl.BlockSpec((BLK, IN_CH), lambda t, b: (b, 0)),
            pl.BlockSpec((1, 1, BLK), lambda t, b: (b, 0, 0)),
            pl.BlockSpec((OUT_CH, 4 * IN_CH), lambda t, b: (0, 0)),
            pl.BlockSpec((IN_CH, 4 * IN_CH), lambda t, b: (0, 0)),
            pl.BlockSpec((1, 4 * IN_CH), lambda t, b: (0, 0)),
        ],
        out_specs=pl.BlockSpec((B, OUT_CH), lambda t, b: (0, 0)),
        out_shape=jax.ShapeDtypeStruct((B, OUT_CH), jnp.float32),
        scratch_shapes=[
            pltpu.VMEM((B, IN_CH), jnp.float32),
            pltpu.VMEM((B, IN_CH), jnp.float32),
            pltpu.VMEM((B, OUT_CH), jnp.float32),
            pltpu.VMEM((B, 1), jnp.float32),
            pltpu.VMEM((B, 1), jnp.float32),
            pltpu.VMEM((B, IN_CH), jnp.float32),
        ],
        compiler_params=pltpu.CompilerParams(
            dimension_semantics=("arbitrary", "arbitrary"),
        ),
    )(x, idx3, wi, wh, bias)


# speedup vs baseline: 33.2107x; 1.2739x over previous
import jax
import jax.numpy as jnp
from jax import lax
from jax.experimental import pallas as pl
from jax.experimental.pallas import tpu as pltpu

N = 100000
IN_CH = 128
OUT_CH = 2 * IN_CH
B = 32
STEPS = 3
BLK = 10000
NB = N // BLK

_NEG = -1e30


def _body(x_ref, idx_ref, wi_ref, wh_ref, b_ref, out_ref,
          h_ref, c_ref, qs_ref, m_ref, s_ref, r_ref):
    t = pl.program_id(0)
    b = pl.program_id(1)

    @pl.when(b == 0)
    def _head():
        @pl.when(t == 0)
        def _init():
            h_ref[...] = jnp.zeros((B, IN_CH), jnp.float32)
            c_ref[...] = jnp.zeros((B, IN_CH), jnp.float32)
            qs_ref[...] = jnp.zeros((B, OUT_CH), jnp.float32)

        @pl.when(t > 0)
        def _finalize_prev():
            r = jnp.where(m_ref[...] > _NEG,
                          r_ref[...] / (s_ref[...] + 1e-16), 0.0)
            qs_ref[:, :IN_CH] = h_ref[...]
            qs_ref[:, IN_CH:] = r

        qs = qs_ref[...]
        gates = (jnp.dot(qs, wi_ref[...], preferred_element_type=jnp.float32)
                 + jnp.dot(h_ref[...], wh_ref[...], preferred_element_type=jnp.float32)
                 + b_ref[...])
        ig = jax.nn.sigmoid(gates[:, :IN_CH])
        fg = jax.nn.sigmoid(gates[:, IN_CH:2 * IN_CH])
        gg = jnp.tanh(gates[:, 2 * IN_CH:3 * IN_CH])
        og = jax.nn.sigmoid(gates[:, 3 * IN_CH:])
        c_new = fg * c_ref[...] + ig * gg
        h_new = og * jnp.tanh(c_new)
        c_ref[...] = c_new
        h_ref[...] = h_new
        m_ref[...] = jnp.full((B, 1), _NEG, jnp.float32)
        s_ref[...] = jnp.zeros((B, 1), jnp.float32)
        r_ref[...] = jnp.zeros((B, IN_CH), jnp.float32)

    xb = x_ref[...]
    idx = idx_ref[0]
    q = h_ref[...]

    seg = lax.broadcasted_iota(jnp.int32, (B, BLK), 0)
    mask = idx == seg

    et = lax.dot_general(q, xb, (((1,), (1,)), ((), ())),
                         preferred_element_type=jnp.float32)
    em = jnp.where(mask, et, _NEG)

    m_old = m_ref[...]
    m_blk = jnp.max(em, axis=1, keepdims=True)
    m_new = jnp.maximum(m_old, m_blk)
    scale = jnp.exp(m_old - m_new)

    p = jnp.exp(em - m_new)

    m_ref[...] = m_new
    s_ref[...] = s_ref[...] * scale + jnp.sum(p, axis=1, keepdims=True)
    r_ref[...] = r_ref[...] * scale + lax.dot_general(
        p, xb, (((1,), (0,)), ((), ())), preferred_element_type=jnp.float32)

    @pl.when((t == STEPS - 1) & (b == NB - 1))
    def _emit():
        out_ref[:, :IN_CH] = h_ref[...]
        out_ref[:, IN_CH:] = jnp.where(m_ref[...] > _NEG,
                                       r_ref[...] / (s_ref[...] + 1e-16), 0.0)


def kernel(x, batch_index, W_ih, W_hh, b_ih, b_hh):
    wi = W_ih.T
    wh = W_hh.T
    bias = (b_ih + b_hh).reshape(1, 4 * IN_CH)
    idx3 = batch_index.reshape(NB, 1, BLK)

    return pl.pallas_call(
        _body,
        grid=(STEPS, NB),
        in_specs=[
            pl.BlockSpec((BLK, IN_CH), lambda t, b: (b, 0)),
            pl.BlockSpec((1, 1, BLK), lambda t, b: (b, 0, 0)),
            pl.BlockSpec((OUT_CH, 4 * IN_CH), lambda t, b: (0, 0)),
            pl.BlockSpec((IN_CH, 4 * IN_CH), lambda t, b: (0, 0)),
            pl.BlockSpec((1, 4 * IN_CH), lambda t, b: (0, 0)),
        ],
        out_specs=pl.BlockSpec((B, OUT_CH), lambda t, b: (0, 0)),
        out_shape=jax.ShapeDtypeStruct((B, OUT_CH), jnp.float32),
        scratch_shapes=[
            pltpu.VMEM((B, IN_CH), jnp.float32),
            pltpu.VMEM((B, IN_CH), jnp.float32),
            pltpu.VMEM((B, OUT_CH), jnp.float32),
            pltpu.VMEM((B, 1), jnp.float32),
            pltpu.VMEM((B, 1), jnp.float32),
            pltpu.VMEM((B, IN_CH), jnp.float32),
        ],
        compiler_params=pltpu.CompilerParams(
            dimension_semantics=("arbitrary", "arbitrary"),
        ),
    )(x, idx3, wi, wh, bias)


# speedup vs baseline: 38.6780x; 1.1646x over previous
import jax
import jax.numpy as jnp
from jax import lax
from jax.experimental import pallas as pl
from jax.experimental.pallas import tpu as pltpu

N = 100000
IN_CH = 128
OUT_CH = 2 * IN_CH
B = 32
STEPS = 3
BLK = 25000
NB = N // BLK

_NEG = -1e30


def _body(x_ref, idx_ref, wi_ref, wh_ref, b_ref, out_ref,
          h_ref, c_ref, qs_ref, m_ref, s_ref, r_ref):
    t = pl.program_id(0)
    b = pl.program_id(1)

    @pl.when(b == 0)
    def _head():
        @pl.when(t == 0)
        def _init():
            h_ref[...] = jnp.zeros((B, IN_CH), jnp.float32)
            c_ref[...] = jnp.zeros((B, IN_CH), jnp.float32)
            qs_ref[...] = jnp.zeros((B, OUT_CH), jnp.float32)

        @pl.when(t > 0)
        def _finalize_prev():
            r = jnp.where(m_ref[...] > _NEG,
                          r_ref[...] / (s_ref[...] + 1e-16), 0.0)
            qs_ref[:, :IN_CH] = h_ref[...]
            qs_ref[:, IN_CH:] = r

        qs = qs_ref[...]
        gates = (jnp.dot(qs, wi_ref[...], preferred_element_type=jnp.float32)
                 + jnp.dot(h_ref[...], wh_ref[...], preferred_element_type=jnp.float32)
                 + b_ref[...])
        ig = jax.nn.sigmoid(gates[:, :IN_CH])
        fg = jax.nn.sigmoid(gates[:, IN_CH:2 * IN_CH])
        gg = jnp.tanh(gates[:, 2 * IN_CH:3 * IN_CH])
        og = jax.nn.sigmoid(gates[:, 3 * IN_CH:])
        c_new = fg * c_ref[...] + ig * gg
        h_new = og * jnp.tanh(c_new)
        c_ref[...] = c_new
        h_ref[...] = h_new
        m_ref[...] = jnp.full((B, 1), _NEG, jnp.float32)
        s_ref[...] = jnp.zeros((B, 1), jnp.float32)
        r_ref[...] = jnp.zeros((B, IN_CH), jnp.float32)

    xb = x_ref[...]
    idx = idx_ref[0]
    q = h_ref[...]

    seg = lax.broadcasted_iota(jnp.int32, (B, BLK), 0)
    mask = idx == seg

    et = lax.dot_general(q, xb, (((1,), (1,)), ((), ())),
                         preferred_element_type=jnp.float32)
    em = jnp.where(mask, et, _NEG)

    m_old = m_ref[...]
    m_blk = jnp.max(em, axis=1, keepdims=True)
    m_new = jnp.maximum(m_old, m_blk)
    scale = jnp.exp(m_old - m_new)

    p = jnp.exp(em - m_new)

    m_ref[...] = m_new
    s_ref[...] = s_ref[...] * scale + jnp.sum(p, axis=1, keepdims=True)
    r_ref[...] = r_ref[...] * scale + lax.dot_general(
        p, xb, (((1,), (0,)), ((), ())), preferred_element_type=jnp.float32)

    @pl.when((t == STEPS - 1) & (b == NB - 1))
    def _emit():
        out_ref[:, :IN_CH] = h_ref[...]
        out_ref[:, IN_CH:] = jnp.where(m_ref[...] > _NEG,
                                       r_ref[...] / (s_ref[...] + 1e-16), 0.0)


def kernel(x, batch_index, W_ih, W_hh, b_ih, b_hh):
    wi = W_ih.T
    wh = W_hh.T
    bias = (b_ih + b_hh).reshape(1, 4 * IN_CH)
    idx3 = batch_index.reshape(NB, 1, BLK)

    return pl.pallas_call(
        _body,
        grid=(STEPS, NB),
        in_specs=[
            pl.BlockSpec((BLK, IN_CH), lambda t, b: (b, 0)),
            pl.BlockSpec((1, 1, BLK), lambda t, b: (b, 0, 0)),
            pl.BlockSpec((OUT_CH, 4 * IN_CH), lambda t, b: (0, 0)),
            pl.BlockSpec((IN_CH, 4 * IN_CH), lambda t, b: (0, 0)),
            pl.BlockSpec((1, 4 * IN_CH), lambda t, b: (0, 0)),
        ],
        out_specs=pl.BlockSpec((B, OUT_CH), lambda t, b: (0, 0)),
        out_shape=jax.ShapeDtypeStruct((B, OUT_CH), jnp.float32),
        scratch_shapes=[
            pltpu.VMEM((B, IN_CH), jnp.float32),
            pltpu.VMEM((B, IN_CH), jnp.float32),
            pltpu.VMEM((B, OUT_CH), jnp.float32),
            pltpu.VMEM((B, 1), jnp.float32),
            pltpu.VMEM((B, 1), jnp.float32),
            pltpu.VMEM((B, IN_CH), jnp.float32),
        ],
        compiler_params=pltpu.CompilerParams(
            dimension_semantics=("arbitrary", "arbitrary"),
        ),
    )(x, idx3, wi, wh, bias)
